# Initial kernel scaffold; baseline (speedup 1.0000x reference)
import jax
import jax.numpy as jnp
from jax.experimental import pallas as pl


def kernel(features_tensor, outputs):
    raise NotImplementedError("write your pallas kernel here")


# speedup vs baseline: 63.6528x; 63.6528x over previous
import functools
import math

import numpy as np
import jax
import jax.numpy as jnp
from jax import lax
from jax.experimental import pallas as pl
from jax.experimental.pallas import tpu as pltpu
from jax.experimental.pallas import tpu_sc as plsc

_H, _W, _C = 270, 480, 512
_HW = _H * _W
_NS = 32400
_NSP = 32768
_NB = 9
_NCLS = 19
_FULL = 1080 * 1920

_BQ = 128
_BK = 2048
_NQ = 256
_NK = _NSP // _BK
_CLS_PAD = 131072
_SENT = 2e30

_SC_TILES = 32
_D_CHUNK = 4064
_D_TOTAL = _D_CHUNK * _SC_TILES


def _index_constants():
    perm = jax.random.permutation(jax.random.key(12345), _HW)
    sidx = jnp.sort(perm[:_NS]).astype(jnp.int32)
    sidx_pad = jnp.zeros((_NSP,), jnp.int32).at[:_NS].set(sidx)
    bias = jnp.where(jnp.arange(_NSP) < _NS, 0.0, 1e30)
    bias = bias.astype(jnp.float32).reshape(_NK, 1, _BK)
    inv = jnp.full((_D_TOTAL,), _NS, jnp.int32)
    inv = inv.at[sidx].set(jnp.arange(_NS, dtype=jnp.int32))
    return sidx_pad, bias, inv


def _argmax_body(x_ref, o_ref):
    x = x_ref[...]
    m = jnp.max(x, axis=0)
    iot = lax.broadcasted_iota(jnp.int32, x.shape, 0).astype(jnp.float32)
    o_ref[0, 0, :] = jnp.min(jnp.where(x == m[None, :], iot, 99.0), axis=0)


def _run_argmax(logits2d):
    nblk = _CLS_PAD // _BK
    out = pl.pallas_call(
        _argmax_body,
        grid=(nblk,),
        in_specs=[pl.BlockSpec((_NCLS, _BK), lambda i: (0, i))],
        out_specs=pl.BlockSpec((1, 1, _BK), lambda i: (i, 0, 0)),
        out_shape=jax.ShapeDtypeStruct((nblk, 1, _BK), jnp.float32),
    )(logits2d)
    return out.reshape(_CLS_PAD)


def _feat_gather_kernel(feats_hbm, sidx_hbm, out_hbm, idx_v, rows_v, sem):
    wid = lax.axis_index("s") * 2 + lax.axis_index("c")
    base = wid * (_NSP // _SC_TILES)
    for c in range(8):
        off = base + c * 128
        pltpu.sync_copy(sidx_hbm.at[pl.ds(off, 128)], idx_v)
        pltpu.async_copy(feats_hbm.at[idx_v], rows_v, sem).wait()
        pltpu.sync_copy(rows_v, out_hbm.at[pl.ds(off, 128)])


def _run_feat_gather(feats_hw, sidx_pad):
    mesh = plsc.VectorSubcoreMesh(core_axis_name="c", subcore_axis_name="s")
    k = pl.kernel(
        _feat_gather_kernel,
        mesh=mesh,
        out_type=jax.ShapeDtypeStruct((_NSP, _C), jnp.float32),
        scratch_types=[
            pltpu.VMEM((128,), jnp.int32),
            pltpu.VMEM((128, _C), jnp.float32),
            pltpu.SemaphoreType.DMA,
        ],
    )
    return k(feats_hw, sidx_pad)


def _cls_gather_kernel(cls_hbm, sidx_hbm, out_hbm, table_v, idx_v, outb_v):
    wid = lax.axis_index("s") * 2 + lax.axis_index("c")
    base = wid * (_NSP // _SC_TILES)
    pltpu.sync_copy(cls_hbm.at[pl.ds(0, 129616)], table_v)
    for c in range(4):
        off = base + c * 256
        pltpu.sync_copy(sidx_hbm.at[pl.ds(off, 256)], idx_v)
        for i in range(16):
            iv = idx_v[pl.ds(i * 16, 16)]
            outb_v[pl.ds(i * 16, 16)] = plsc.load_gather(table_v, [iv])
        pltpu.sync_copy(outb_v, out_hbm.at[pl.ds(off, 256)])


def _run_cls_gather(cls_flat, sidx_pad):
    mesh = plsc.VectorSubcoreMesh(core_axis_name="c", subcore_axis_name="s")
    k = pl.kernel(
        _cls_gather_kernel,
        mesh=mesh,
        out_type=jax.ShapeDtypeStruct((_NSP,), jnp.float32),
        scratch_types=[
            pltpu.VMEM((129616,), jnp.float32),
            pltpu.VMEM((256,), jnp.int32),
            pltpu.VMEM((256,), jnp.float32),
        ],
        compiler_params=pltpu.CompilerParams(needs_layout_passes=False),
    )
    return k(cls_flat, sidx_pad)


def _main_body(q_ref, k_ref, bias_ref, cls_ref, o_ref, d_ref):
    kk = pl.program_id(1)
    q = q_ref[...]
    kb = k_ref[...]
    qn = jnp.sum(q * q, axis=1)
    kn = jnp.sum(kb * kb, axis=1)
    dot = lax.dot_general(q, kb, (((1,), (1,)), ((), ())),
                          preferred_element_type=jnp.float32)
    d = -2.0 * dot + qn[:, None] + kn[None, :] + bias_ref[0, 0, :][None, :]
    d = jnp.maximum(d, 1e-12)
    d_ref[:, pl.ds(pl.multiple_of(kk * _BK, _BK), _BK)] = d

    @pl.when(kk == _NK - 1)
    def _finish():
        def rowmin():
            m = None
            for j in range(_NK):
                mj = jnp.min(d_ref[:, j * _BK:(j + 1) * _BK], axis=1)
                m = mj if m is None else jnp.minimum(m, mj)
            return m

        def round_body(_, carry):
            v = rowmin()
            for j in range(_NK):
                sl = slice(j * _BK, (j + 1) * _BK)
                ch = d_ref[:, sl]
                d_ref[:, sl] = jnp.where(ch == v[:, None], _SENT, ch)
            return carry

        lax.fori_loop(0, _NB - 1, round_body, jnp.float32(0.0))
        t9 = rowmin()
        hist = jnp.zeros((_BQ, _NCLS), jnp.float32)
        for j in range(_NK):
            sl = slice(j * _BK, (j + 1) * _BK)
            ch = d_ref[:, sl]
            act = jnp.logical_or(ch <= t9[:, None], ch == _SENT)
            hist = hist + lax.dot_general(
                act.astype(jnp.float32), cls_ref[:, sl],
                (((1,), (1,)), ((), ())), preferred_element_type=jnp.float32)
        total = jnp.sum(hist, axis=1)
        p = hist / total[:, None]
        ent = jnp.sum(-p * jnp.log(p + 1e-6), axis=1) / math.log(float(_NCLS))
        o_ref[0, 0, :] = ent


def _run_main(gathered, bias, cls1ht):
    out = pl.pallas_call(
        _main_body,
        grid=(_NQ, _NK),
        in_specs=[
            pl.BlockSpec((_BQ, _C), lambda qi, ki: (qi, 0)),
            pl.BlockSpec((_BK, _C), lambda qi, ki: (ki, 0)),
            pl.BlockSpec((1, 1, _BK), lambda qi, ki: (ki, 0, 0)),
            pl.BlockSpec((_NCLS, _NSP), lambda qi, ki: (0, 0)),
        ],
        out_specs=pl.BlockSpec((1, 1, _BQ), lambda qi, ki: (qi, 0, 0)),
        out_shape=jax.ShapeDtypeStruct((_NQ, 1, _BQ), jnp.float32),
        scratch_shapes=[pltpu.VMEM((_BQ, _NSP), jnp.float32)],
        compiler_params=pltpu.CompilerParams(
            dimension_semantics=("arbitrary", "arbitrary")),
    )(gathered, gathered, bias, cls1ht)
    return out.reshape(_NQ * _BQ)


def _scatter_kernel(ptab_hbm, inv_hbm, out_hbm, table_v, inv_v, outb_v):
    wid = lax.axis_index("s") * 2 + lax.axis_index("c")
    base = wid * _D_CHUNK
    pltpu.sync_copy(ptab_hbm, table_v)
    pltpu.sync_copy(inv_hbm.at[pl.ds(base, _D_CHUNK)], inv_v)

    def body(i, carry):
        iv = inv_v[pl.ds(i * 16, 16)]
        outb_v[pl.ds(i * 16, 16)] = plsc.load_gather(table_v, [iv])
        return carry

    lax.fori_loop(0, _D_CHUNK // 16, body, jnp.float32(0.0))
    pltpu.sync_copy(outb_v, out_hbm.at[pl.ds(base, _D_CHUNK)])


def _run_scatter(ptab, inv):
    mesh = plsc.VectorSubcoreMesh(core_axis_name="c", subcore_axis_name="s")
    k = pl.kernel(
        _scatter_kernel,
        mesh=mesh,
        out_type=jax.ShapeDtypeStruct((_D_TOTAL,), jnp.float32),
        scratch_types=[
            pltpu.VMEM((_NS + 16,), jnp.float32),
            pltpu.VMEM((_D_CHUNK,), jnp.int32),
            pltpu.VMEM((_D_CHUNK,), jnp.float32),
        ],
        compiler_params=pltpu.CompilerParams(needs_layout_passes=False),
    )
    return k(ptab, inv)


def kernel(features_tensor, outputs):
    sidx_pad, bias, inv = _index_constants()

    feats_hw = jnp.transpose(features_tensor[0], (1, 2, 0)).reshape(_HW, _C)
    logits2d = jnp.pad(outputs[0].reshape(_NCLS, _HW),
                       ((0, 0), (0, _CLS_PAD - _HW)))

    cls_flat = _run_argmax(logits2d)
    gathered = _run_feat_gather(feats_hw, sidx_pad)
    cls_s = _run_cls_gather(cls_flat, sidx_pad)

    cls1ht = (jnp.arange(_NCLS, dtype=jnp.float32)[:, None]
              == cls_s[None, :]).astype(jnp.float32)

    purity = _run_main(gathered, bias, cls1ht)
    ptab = jnp.concatenate([purity[:_NS], jnp.zeros((16,), jnp.float32)])

    scat = _run_scatter(ptab, inv)
    full = jnp.concatenate(
        [scat[:_HW], jnp.zeros((_FULL - _HW,), jnp.float32)])
    return full.reshape(1080, 1920)


# speedup vs baseline: 86.1475x; 1.3534x over previous
import functools
import math

import numpy as np
import jax
import jax.numpy as jnp
from jax import lax
from jax.experimental import pallas as pl
from jax.experimental.pallas import tpu as pltpu
from jax.experimental.pallas import tpu_sc as plsc

_H, _W, _C = 270, 480, 512
_HW = _H * _W
_NS = 32400
_NSP = 32768
_NB = 9
_NCLS = 19
_FULL = 1080 * 1920

_BQ = 256
_BK = 2048
_NQ = 128
_NK = _NSP // _BK
_CLS_PAD = 131072
_SENT = 2e30

_SC_TILES = 32
_D_CHUNK = 4064
_D_TOTAL = _D_CHUNK * _SC_TILES


def _index_constants():
    perm = jax.random.permutation(jax.random.key(12345), _HW)
    sidx = jnp.sort(perm[:_NS]).astype(jnp.int32)
    sidx_pad = jnp.zeros((_NSP,), jnp.int32).at[:_NS].set(sidx)
    bias = jnp.where(jnp.arange(_NSP) < _NS, 0.0, 1e30)
    bias = bias.astype(jnp.float32).reshape(_NK, 1, _BK)
    inv = jnp.full((_D_TOTAL,), _NS, jnp.int32)
    inv = inv.at[sidx].set(jnp.arange(_NS, dtype=jnp.int32))
    return sidx_pad, bias, inv


def _argmax_body(x_ref, o_ref):
    x = x_ref[...]
    m = jnp.max(x, axis=0)
    iot = lax.broadcasted_iota(jnp.int32, x.shape, 0).astype(jnp.float32)
    o_ref[0, 0, :] = jnp.min(jnp.where(x == m[None, :], iot, 99.0), axis=0)


def _run_argmax(logits2d):
    nblk = _CLS_PAD // _BK
    out = pl.pallas_call(
        _argmax_body,
        grid=(nblk,),
        in_specs=[pl.BlockSpec((_NCLS, _BK), lambda i: (0, i))],
        out_specs=pl.BlockSpec((1, 1, _BK), lambda i: (i, 0, 0)),
        out_shape=jax.ShapeDtypeStruct((nblk, 1, _BK), jnp.float32),
    )(logits2d)
    return out.reshape(_CLS_PAD)


def _feat_gather_kernel(feats_hbm, sidx_hbm, out_hbm, idx_v, rows_v, sem):
    wid = lax.axis_index("s") * 2 + lax.axis_index("c")
    base = wid * (_NSP // _SC_TILES)
    for c in range(8):
        off = base + c * 128
        pltpu.sync_copy(sidx_hbm.at[pl.ds(off, 128)], idx_v)
        pltpu.async_copy(feats_hbm.at[idx_v], rows_v, sem).wait()
        pltpu.sync_copy(rows_v, out_hbm.at[pl.ds(off, 128)])


def _run_feat_gather(feats_hw, sidx_pad):
    mesh = plsc.VectorSubcoreMesh(core_axis_name="c", subcore_axis_name="s")
    k = pl.kernel(
        _feat_gather_kernel,
        mesh=mesh,
        out_type=jax.ShapeDtypeStruct((_NSP, _C), jnp.float32),
        scratch_types=[
            pltpu.VMEM((128,), jnp.int32),
            pltpu.VMEM((128, _C), jnp.float32),
            pltpu.SemaphoreType.DMA,
        ],
    )
    return k(feats_hw, sidx_pad)


def _cls_gather_kernel(cls_hbm, sidx_hbm, out_hbm, table_v, idx_v, outb_v):
    wid = lax.axis_index("s") * 2 + lax.axis_index("c")
    base = wid * (_NSP // _SC_TILES)
    pltpu.sync_copy(cls_hbm.at[pl.ds(0, 129616)], table_v)
    for c in range(4):
        off = base + c * 256
        pltpu.sync_copy(sidx_hbm.at[pl.ds(off, 256)], idx_v)
        for i in range(16):
            iv = idx_v[pl.ds(i * 16, 16)]
            outb_v[pl.ds(i * 16, 16)] = plsc.load_gather(table_v, [iv])
        pltpu.sync_copy(outb_v, out_hbm.at[pl.ds(off, 256)])


def _run_cls_gather(cls_flat, sidx_pad):
    mesh = plsc.VectorSubcoreMesh(core_axis_name="c", subcore_axis_name="s")
    k = pl.kernel(
        _cls_gather_kernel,
        mesh=mesh,
        out_type=jax.ShapeDtypeStruct((_NSP,), jnp.float32),
        scratch_types=[
            pltpu.VMEM((129616,), jnp.float32),
            pltpu.VMEM((256,), jnp.int32),
            pltpu.VMEM((256,), jnp.float32),
        ],
        compiler_params=pltpu.CompilerParams(needs_layout_passes=False),
    )
    return k(cls_flat, sidx_pad)


def _main_body(q_ref, k_ref, bias_ref, cls_ref, o_ref, d_ref):
    kk = pl.program_id(1)
    q = q_ref[...]
    kb = k_ref[...]
    qn = jnp.sum(q * q, axis=1)
    kn = jnp.sum(kb * kb, axis=1)
    dot = lax.dot_general(q, kb, (((1,), (1,)), ((), ())),
                          preferred_element_type=jnp.float32)
    d = -2.0 * dot + qn[:, None] + kn[None, :] + bias_ref[0, 0, :][None, :]
    d = jnp.maximum(d, 1e-12)
    d_ref[:, pl.ds(pl.multiple_of(kk * _BK, _BK), _BK)] = d

    @pl.when(kk == _NK - 1)
    def _finish():
        def rowmin():
            m = None
            for j in range(_NK):
                mj = jnp.min(d_ref[:, j * _BK:(j + 1) * _BK], axis=1)
                m = mj if m is None else jnp.minimum(m, mj)
            return m

        def round_body(_, carry):
            v = rowmin()
            for j in range(_NK):
                sl = slice(j * _BK, (j + 1) * _BK)
                ch = d_ref[:, sl]
                d_ref[:, sl] = jnp.where(ch == v[:, None], _SENT, ch)
            return carry

        lax.fori_loop(0, _NB - 1, round_body, jnp.float32(0.0))
        t9 = rowmin()
        hist = jnp.zeros((_BQ, _NCLS), jnp.float32)
        for j in range(_NK):
            sl = slice(j * _BK, (j + 1) * _BK)
            ch = d_ref[:, sl]
            act = jnp.logical_or(ch <= t9[:, None], ch == _SENT)
            hist = hist + lax.dot_general(
                act.astype(jnp.float32), cls_ref[:, sl],
                (((1,), (1,)), ((), ())), preferred_element_type=jnp.float32)
        total = jnp.sum(hist, axis=1)
        p = hist / total[:, None]
        ent = jnp.sum(-p * jnp.log(p + 1e-6), axis=1) / math.log(float(_NCLS))
        o_ref[0, 0, :] = ent


def _run_main(gathered, bias, cls1ht):
    out = pl.pallas_call(
        _main_body,
        grid=(_NQ, _NK),
        in_specs=[
            pl.BlockSpec((_BQ, _C), lambda qi, ki: (qi, 0)),
            pl.BlockSpec((_BK, _C), lambda qi, ki: (ki, 0)),
            pl.BlockSpec((1, 1, _BK), lambda qi, ki: (ki, 0, 0)),
            pl.BlockSpec((_NCLS, _NSP), lambda qi, ki: (0, 0)),
        ],
        out_specs=pl.BlockSpec((1, 1, _BQ), lambda qi, ki: (qi, 0, 0)),
        out_shape=jax.ShapeDtypeStruct((_NQ, 1, _BQ), jnp.float32),
        scratch_shapes=[pltpu.VMEM((_BQ, _NSP), jnp.float32)],
        compiler_params=pltpu.CompilerParams(
            dimension_semantics=("arbitrary", "arbitrary")),
    )(gathered, gathered, bias, cls1ht)
    return out.reshape(_NQ * _BQ)


def _scatter_kernel(ptab_hbm, inv_hbm, out_hbm, table_v, inv_v, outb_v):
    wid = lax.axis_index("s") * 2 + lax.axis_index("c")
    base = wid * _D_CHUNK
    pltpu.sync_copy(ptab_hbm, table_v)
    pltpu.sync_copy(inv_hbm.at[pl.ds(base, _D_CHUNK)], inv_v)

    def body(i, carry):
        iv = inv_v[pl.ds(i * 16, 16)]
        outb_v[pl.ds(i * 16, 16)] = plsc.load_gather(table_v, [iv])
        return carry

    lax.fori_loop(0, _D_CHUNK // 16, body, jnp.float32(0.0))
    pltpu.sync_copy(outb_v, out_hbm.at[pl.ds(base, _D_CHUNK)])


def _run_scatter(ptab, inv):
    mesh = plsc.VectorSubcoreMesh(core_axis_name="c", subcore_axis_name="s")
    k = pl.kernel(
        _scatter_kernel,
        mesh=mesh,
        out_type=jax.ShapeDtypeStruct((_D_TOTAL,), jnp.float32),
        scratch_types=[
            pltpu.VMEM((_NS + 16,), jnp.float32),
            pltpu.VMEM((_D_CHUNK,), jnp.int32),
            pltpu.VMEM((_D_CHUNK,), jnp.float32),
        ],
        compiler_params=pltpu.CompilerParams(needs_layout_passes=False),
    )
    return k(ptab, inv)


def kernel(features_tensor, outputs):
    sidx_pad, bias, inv = _index_constants()

    feats_hw = jnp.transpose(features_tensor[0], (1, 2, 0)).reshape(_HW, _C)
    logits2d = jnp.pad(outputs[0].reshape(_NCLS, _HW),
                       ((0, 0), (0, _CLS_PAD - _HW)))

    cls_flat = _run_argmax(logits2d)
    gathered = _run_feat_gather(feats_hw, sidx_pad)
    cls_s = _run_cls_gather(cls_flat, sidx_pad)

    cls1ht = (jnp.arange(_NCLS, dtype=jnp.float32)[:, None]
              == cls_s[None, :]).astype(jnp.float32)

    purity = _run_main(gathered, bias, cls1ht)
    ptab = jnp.concatenate([purity[:_NS], jnp.zeros((16,), jnp.float32)])

    scat = _run_scatter(ptab, inv)
    full = jnp.concatenate(
        [scat[:_HW], jnp.zeros((_FULL - _HW,), jnp.float32)])
    return full.reshape(1080, 1920)
